# baseline (speedup 1.0000x reference)
import functools

import jax
import jax.numpy as jnp
from jax import lax
from jax.experimental import pallas as pl
from jax.experimental.pallas import tpu as pltpu
from jax.experimental.pallas import tpu_sc as plsc

_B, _L, _V, _D, _C = 4096, 50, 1000000, 64, 5
_N = _B * _L
_NW = 32
_ROWS_PER_W = _N // _NW
_CHUNK = 128
_CHUNKS = _ROWS_PER_W // _CHUNK
_NBUF = 5

_CPAD = 8


def _gather_body(word_hbm, table_hbm, out_hbm, idx_v, buf_v, gsem):
    cid = lax.axis_index("c")
    sid = lax.axis_index("s")
    wid = sid * 2 + cid
    base = wid * _ROWS_PER_W
    pltpu.sync_copy(word_hbm.at[wid], idx_v)

    def fire(j, slot):
        pltpu.async_copy(table_hbm.at[idx_v.at[j]], buf_v.at[slot], gsem.at[slot])

    for s in range(_NBUF):
        fire(s, s)

    def outer(j0, carry):
        for s in range(_NBUF):
            j = j0 * _NBUF + s
            pltpu.make_async_copy(
                table_hbm.at[idx_v.at[j]], buf_v.at[s], gsem.at[s]
            ).wait()
            pltpu.sync_copy(buf_v.at[s], out_hbm.at[pl.ds(base + j * _CHUNK, _CHUNK)])

            @pl.when(j + _NBUF < _CHUNKS)
            def _():
                fire(j + _NBUF, s)

        return carry

    lax.fori_loop(0, _CHUNKS // _NBUF, outer, 0)


@functools.cache
def _sc_gather_fn():
    return pl.kernel(
        _gather_body,
        out_type=jax.ShapeDtypeStruct((_N, _D), jnp.float32),
        mesh=plsc.VectorSubcoreMesh(core_axis_name="c", subcore_axis_name="s"),
        scratch_types=[
            pltpu.VMEM((_CHUNKS, _CHUNK), jnp.int32),
            pltpu.VMEM((_NBUF, _CHUNK, _D), jnp.float32),
            pltpu.SemaphoreType.DMA((_NBUF,)),
        ],
        compiler_params=pltpu.CompilerParams(use_tc_tiling_on_sc=False),
    )


def _tc_body(x_ref, w_ref, b_ref, out_ref):
    bs = x_ref.shape[0]
    x2 = x_ref[...].reshape(bs * _L, _D)
    p = jnp.dot(x2, w_ref[...], preferred_element_type=jnp.float32)
    p = p.reshape(bs, _L, 5 * _CPAD)
    z2 = jnp.zeros((bs, 2, 5 * _CPAD), jnp.float32)
    ppad = jnp.concatenate([z2, p[:, : _L - 2], z2, p[:, _L - 2 :]], axis=1)
    acc = ppad[:, 0:_L, 0:_C]
    for i in range(1, 5):
        acc = acc + ppad[:, i : i + _L, i * _CPAD : i * _CPAD + _C]
    out_ref[...] = acc + b_ref[...]


def _tc_project(x, wstack, bias):
    bs = 256
    grid = _B // bs
    return pl.pallas_call(
        _tc_body,
        grid=(grid,),
        in_specs=[
            pl.BlockSpec((bs, _L, _D), lambda i: (i, 0, 0)),
            pl.BlockSpec((_D, 5 * _CPAD), lambda i: (0, 0)),
            pl.BlockSpec((1, 1, _C), lambda i: (0, 0, 0)),
        ],
        out_specs=pl.BlockSpec((bs, _L, _C), lambda i: (i, 0, 0)),
        out_shape=jax.ShapeDtypeStruct((_B, _L, _C), jnp.float32),
    )(x, wstack, bias)


def kernel(word, embed_table, W, b):
    word_i32 = word.astype(jnp.int32).reshape(_NW, _CHUNKS, _CHUNK)
    x = _sc_gather_fn()(word_i32, embed_table)
    w_chunks = W.reshape(_C, 5, _D)
    wstack = jnp.zeros((_D, 5, _CPAD), jnp.float32)
    wstack = wstack.at[:, :, :_C].set(jnp.transpose(w_chunks, (2, 1, 0)))
    wstack = wstack.reshape(_D, 5 * _CPAD)
    logit = _tc_project(x.reshape(_B, _L, _D), wstack, b.reshape(1, 1, _C))
    return logit


# speedup vs baseline: 1.0903x; 1.0903x over previous
import functools

import jax
import jax.numpy as jnp
from jax import lax
from jax.experimental import pallas as pl
from jax.experimental.pallas import tpu as pltpu
from jax.experimental.pallas import tpu_sc as plsc

_B, _L, _V, _D, _C = 4096, 50, 1000000, 64, 5
_N = _B * _L
_NW = 32
_ROWS_PER_W = _N // _NW
_CHUNK = 128
_CHUNKS = _ROWS_PER_W // _CHUNK
_NBUF = 5

_CPAD = 8


def _gather_body(word_hbm, table_hbm, out_hbm, idx_v, buf_v, gsem):
    cid = lax.axis_index("c")
    sid = lax.axis_index("s")
    wid = sid * 2 + cid
    base = wid * _ROWS_PER_W
    pltpu.sync_copy(word_hbm.at[wid], idx_v)

    def fire(j, slot):
        pltpu.async_copy(table_hbm.at[idx_v.at[j]], buf_v.at[slot], gsem.at[slot])

    for s in range(_NBUF):
        fire(s, s)

    def outer(j0, carry):
        for s in range(_NBUF):
            j = j0 * _NBUF + s
            pltpu.make_async_copy(
                table_hbm.at[idx_v.at[j]], buf_v.at[s], gsem.at[s]
            ).wait()
            pltpu.sync_copy(buf_v.at[s], out_hbm.at[pl.ds(base + j * _CHUNK, _CHUNK)])

            @pl.when(j + _NBUF < _CHUNKS)
            def _():
                fire(j + _NBUF, s)

        return carry

    lax.fori_loop(0, _CHUNKS // _NBUF, outer, 0)


@functools.cache
def _sc_gather_fn():
    return pl.kernel(
        _gather_body,
        out_type=jax.ShapeDtypeStruct((_N, _D), jnp.float32),
        mesh=plsc.VectorSubcoreMesh(core_axis_name="c", subcore_axis_name="s"),
        scratch_types=[
            pltpu.VMEM((_CHUNKS, _CHUNK), jnp.int32),
            pltpu.VMEM((_NBUF, _CHUNK, _D), jnp.float32),
            pltpu.SemaphoreType.DMA((_NBUF,)),
        ],
        compiler_params=pltpu.CompilerParams(use_tc_tiling_on_sc=False),
    )


_LH = _L // 2


def _tc_body(x_ref, w_ref, b_ref, oute_ref, outo_ref):
    bs = oute_ref.shape[2]
    p2 = jnp.dot(x_ref[...], w_ref[...], preferred_element_type=jnp.float32)
    p3 = p2.reshape(bs, _LH, 80)
    pe = p3[:, :, 0:40]
    po = p3[:, :, 40:80]
    z1 = jnp.zeros((bs, 1, 40), jnp.float32)
    ppe = jnp.concatenate([z1, pe[:, 0:24], z1, pe[:, 24:25]], axis=1)
    ppo = jnp.concatenate([z1, po[:, 0:24], z1, po[:, 24:25]], axis=1)
    acc_e = (ppe[:, 0:25, 0:5] + ppo[:, 0:25, 8:13] + ppe[:, 1:26, 16:21]
             + ppo[:, 1:26, 24:29] + ppe[:, 2:27, 32:37]) + b_ref[...]
    acc_o = (ppo[:, 0:25, 0:5] + ppe[:, 1:26, 8:13] + ppo[:, 1:26, 16:21]
             + ppe[:, 2:27, 24:29] + ppo[:, 2:27, 32:37]) + b_ref[...]
    oute_ref[...] = jnp.transpose(acc_e, (2, 1, 0))
    outo_ref[...] = jnp.transpose(acc_o, (2, 1, 0))


def _tc_project(x128, wstack2, bias):
    bs = 256
    grid = _B // bs
    return pl.pallas_call(
        _tc_body,
        grid=(grid,),
        in_specs=[
            pl.BlockSpec((bs * _LH, 128), lambda i: (i, 0)),
            pl.BlockSpec((128, 80), lambda i: (0, 0)),
            pl.BlockSpec((1, 1, _C), lambda i: (0, 0, 0)),
        ],
        out_specs=[
            pl.BlockSpec((_C, _LH, bs), lambda i: (0, 0, i)),
            pl.BlockSpec((_C, _LH, bs), lambda i: (0, 0, i)),
        ],
        out_shape=[
            jax.ShapeDtypeStruct((_C, _LH, _B), jnp.float32),
            jax.ShapeDtypeStruct((_C, _LH, _B), jnp.float32),
        ],
    )(x128, wstack2, bias)


def kernel(word, embed_table, W, b):
    word_i32 = word.astype(jnp.int32).reshape(_NW, _CHUNKS, _CHUNK)
    x = _sc_gather_fn()(word_i32, embed_table)
    w_chunks = W.reshape(_C, 5, _D)
    wstack = jnp.zeros((_D, 5, _CPAD), jnp.float32)
    wstack = wstack.at[:, :, :_C].set(jnp.transpose(w_chunks, (2, 1, 0)))
    wstack = wstack.reshape(_D, 5 * _CPAD)
    wstack2 = jnp.zeros((2 * _D, 80), jnp.float32)
    wstack2 = wstack2.at[:_D, :40].set(wstack)
    wstack2 = wstack2.at[_D:, 40:].set(wstack)
    x128 = x.reshape(_N // 2, 2 * _D)
    oute, outo = _tc_project(x128, wstack2, b.reshape(1, 1, _C))
    out_t = jnp.stack([oute, outo], axis=2).reshape(_C, _L, _B)
    return jnp.transpose(out_t, (2, 1, 0))
